# baseline (speedup 1.0000x reference)
import jax
import jax.numpy as jnp
from jax import lax
from jax.experimental import pallas as pl
from jax.experimental.pallas import tpu as pltpu
from jax.experimental.pallas import tpu_sc as plsc

_B, _S, _D = 4, 8192, 768
_N = 16
_NG = 3
_KC, _KQK, _KV = 8, 4, 6
_TS = 4096
_SC_CORES = 2
_TASKS = _NG * _B


def _stream_body(x_ref, imp_ref, w_ref, g48_ref, dw_ref, acc_ref):
    s = pl.program_id(1)
    ns = pl.num_programs(1)

    @pl.when(s == 0)
    def _init():
        acc_ref[...] = jnp.zeros_like(acc_ref)

    xt = x_ref[0]
    logits = jnp.dot(xt, w_ref[...], preferred_element_type=jnp.float32)
    e = jnp.exp(logits)
    d = jnp.dot(e, g48_ref[...], preferred_element_type=jnp.float32)
    p = e * (1.0 / d)
    imp = imp_ref[0, 0]
    acc_ref[...] += jnp.dot(imp, p, preferred_element_type=jnp.float32)

    @pl.when(s == ns - 1)
    def _finish():
        dw_ref[...] = acc_ref[...].reshape(1, 1, _NG * _N)


def _sc_topk_body(dense_hbm, out_hbm, vin, wbuf, ibuf, rankbuf, tbuf):
    wid = lax.axis_index("s") * _SC_CORES + lax.axis_index("c")

    @pl.when(wid < _TASKS)
    def _():
        g = wid // _B
        b = wid - g * _B
        pltpu.sync_copy(dense_hbm.at[pl.ds((b * _NG + g) * _N, _N)], vin)
        v0 = vin[...]
        klim = jnp.where(g == 0, _KC, jnp.where(g == 1, _KQK, _KV))
        iota = lax.broadcasted_iota(jnp.int32, (_N,), 0)
        _, si = plsc.sort_key_val(v0, iota, descending=True)
        plsc.store_scatter(rankbuf, [si], iota)
        mask = rankbuf[...] < klim
        contrib = jnp.where(mask, v0, 0.0)
        t = contrib
        for sh in (1, 2, 4, 8):
            tbuf[...] = t
            t = t + plsc.load_gather(tbuf, [iota ^ sh])
        wbuf[...] = plsc.bitcast(contrib / (t + 1e-8), jnp.int32)
        ibuf[...] = si
        pltpu.sync_copy(wbuf, out_hbm.at[pl.ds(wid * _N, _N)])
        pltpu.sync_copy(ibuf, out_hbm.at[pl.ds((_TASKS + wid) * _N, _N)])


def kernel(x, importance, Wc, Wqk, Wv):
    ns = _S // _TS
    w = jnp.concatenate([Wc, Wqk, Wv], axis=0).T
    imp = importance.reshape(_B, ns, 1, _TS)
    lanes = jnp.arange(_NG * _N)
    g48 = (lanes[:, None] // _N == lanes[None, :] // _N).astype(jnp.float32)

    dense = pl.pallas_call(
        _stream_body,
        grid=(_B, ns),
        in_specs=[
            pl.BlockSpec((1, _TS, _D), lambda b, s: (b, s, 0)),
            pl.BlockSpec((1, 1, 1, _TS), lambda b, s: (b, s, 0, 0)),
            pl.BlockSpec((_D, _NG * _N), lambda b, s: (0, 0)),
            pl.BlockSpec((_NG * _N, _NG * _N), lambda b, s: (0, 0)),
        ],
        out_specs=pl.BlockSpec((1, 1, _NG * _N), lambda b, s: (b, 0, 0)),
        out_shape=jax.ShapeDtypeStruct((_B, 1, _NG * _N), jnp.float32),
        scratch_shapes=[pltpu.VMEM((1, _NG * _N), jnp.float32)],
        compiler_params=pltpu.CompilerParams(
            dimension_semantics=("parallel", "arbitrary"),
        ),
    )(x, imp, w, g48)

    sc_topk = pl.kernel(
        _sc_topk_body,
        out_type=jax.ShapeDtypeStruct((2 * _TASKS * _N,), jnp.int32),
        mesh=plsc.VectorSubcoreMesh(core_axis_name="c", subcore_axis_name="s"),
        scratch_types=[pltpu.VMEM((_N,), jnp.float32),
                       pltpu.VMEM((_N,), jnp.int32),
                       pltpu.VMEM((_N,), jnp.int32),
                       pltpu.VMEM((_N,), jnp.int32),
                       pltpu.VMEM((_N,), jnp.float32)],
        compiler_params=pltpu.CompilerParams(needs_layout_passes=False),
    )
    out = sc_topk(dense.reshape(_B * _NG * _N))
    w3 = jax.lax.bitcast_convert_type(
        out[:_TASKS * _N], jnp.float32).reshape(_NG, _B, _N)
    i3 = out[_TASKS * _N:].reshape(_NG, _B, _N)
    return (w3[0], w3[1], w3[2],
            i3[0, :, :_KC], i3[1, :, :_KQK], i3[2, :, :_KV])


# speedup vs baseline: 1.0351x; 1.0351x over previous
import jax
import jax.numpy as jnp
from jax import lax
from jax.experimental import pallas as pl
from jax.experimental.pallas import tpu as pltpu
from jax.experimental.pallas import tpu_sc as plsc

_B, _S, _D = 4, 8192, 768
_N = 16
_NG = 3
_KC, _KQK, _KV = 8, 4, 6
_TS = 4096
_SC_CORES = 2
_TASKS = _NG * _B


def _stream_body(x_ref, imp_ref, w_ref, g48_ref, dw_ref, acc_ref):
    s = pl.program_id(1)
    ns = pl.num_programs(1)

    @pl.when(s == 0)
    def _init():
        acc_ref[...] = jnp.zeros_like(acc_ref)

    xt = x_ref[0]
    logits = jnp.dot(xt, w_ref[...], preferred_element_type=jnp.float32)
    e = jnp.exp(logits)
    d = jnp.dot(e, g48_ref[...], preferred_element_type=jnp.float32)
    p = e * (1.0 / d)
    imp = imp_ref[0, 0]
    acc_ref[...] += jnp.dot(imp, p, preferred_element_type=jnp.float32)

    @pl.when(s == ns - 1)
    def _finish():
        dw_ref[...] = acc_ref[...].reshape(1, 1, _NG * _N)


def _sc_topk_body(dense_hbm, out_hbm, vin, wbuf, ibuf, rankbuf, tbuf):
    wid = lax.axis_index("s") * _SC_CORES + lax.axis_index("c")

    @pl.when(wid < _TASKS)
    def _():
        g = wid // _B
        b = wid - g * _B
        pltpu.sync_copy(dense_hbm.at[pl.ds((b * _NG + g) * _N, _N)], vin)
        v0 = vin[...]
        klim = jnp.where(g == 0, _KC, jnp.where(g == 1, _KQK, _KV))
        iota = lax.broadcasted_iota(jnp.int32, (_N,), 0)
        _, si = plsc.sort_key_val(v0, iota, descending=True)
        plsc.store_scatter(rankbuf, [si], iota)
        mask = rankbuf[...] < klim
        contrib = jnp.where(mask, v0, 0.0)
        t = contrib
        for sh in (1, 2, 4, 8):
            tbuf[...] = t
            t = t + plsc.load_gather(tbuf, [iota ^ sh])
        wbuf[...] = plsc.bitcast(contrib / (t + 1e-8), jnp.int32)
        ibuf[...] = si
        pltpu.sync_copy(wbuf, out_hbm.at[pl.ds(wid * _N, _N)])
        pltpu.sync_copy(ibuf, out_hbm.at[pl.ds((_TASKS + wid) * _N, _N)])


def kernel(x, importance, Wc, Wqk, Wv):
    ns = _S // _TS
    w = jnp.concatenate([Wc, Wqk, Wv], axis=0).T
    imp = importance.reshape(_B, ns, 1, _TS)
    lanes = jnp.arange(_NG * _N)
    g48 = (lanes[:, None] // _N == lanes[None, :] // _N).astype(jnp.float32)

    dense = pl.pallas_call(
        _stream_body,
        grid=(_B, ns),
        in_specs=[
            pl.BlockSpec((1, _TS, _D), lambda b, s: (b, s, 0)),
            pl.BlockSpec((1, 1, 1, _TS), lambda b, s: (b, s, 0, 0)),
            pl.BlockSpec((_D, _NG * _N), lambda b, s: (0, 0)),
            pl.BlockSpec((_NG * _N, _NG * _N), lambda b, s: (0, 0)),
        ],
        out_specs=pl.BlockSpec((1, 1, _NG * _N), lambda b, s: (b, 0, 0)),
        out_shape=jax.ShapeDtypeStruct((_B, 1, _NG * _N), jnp.float32),
        scratch_shapes=[pltpu.VMEM((1, _NG * _N), jnp.float32)],
        compiler_params=pltpu.CompilerParams(
            dimension_semantics=("arbitrary", "arbitrary"),
        ),
    )(x, imp, w, g48)

    sc_topk = pl.kernel(
        _sc_topk_body,
        out_type=jax.ShapeDtypeStruct((2 * _TASKS * _N,), jnp.int32),
        mesh=plsc.VectorSubcoreMesh(core_axis_name="c", subcore_axis_name="s"),
        scratch_types=[pltpu.VMEM((_N,), jnp.float32),
                       pltpu.VMEM((_N,), jnp.int32),
                       pltpu.VMEM((_N,), jnp.int32),
                       pltpu.VMEM((_N,), jnp.int32),
                       pltpu.VMEM((_N,), jnp.float32)],
        compiler_params=pltpu.CompilerParams(needs_layout_passes=False),
    )
    out = sc_topk(dense.reshape(_B * _NG * _N))
    w3 = jax.lax.bitcast_convert_type(
        out[:_TASKS * _N], jnp.float32).reshape(_NG, _B, _N)
    i3 = out[_TASKS * _N:].reshape(_NG, _B, _N)
    return (w3[0], w3[1], w3[2],
            i3[0, :, :_KC], i3[1, :, :_KQK], i3[2, :, :_KV])
